# Initial kernel scaffold; baseline (speedup 1.0000x reference)
import jax
import jax.numpy as jnp
from jax.experimental import pallas as pl


def kernel(x, edge_index, edge_attr, batch, W1, root1, b1, W2, root2, b2, Wl1, bl1, Wl2, bl2):
    raise NotImplementedError("write your pallas kernel here")


# speedup vs baseline: 3.6515x; 3.6515x over previous
import functools

import jax
import jax.numpy as jnp
from jax import lax
from jax.experimental import pallas as pl
from jax.experimental.pallas import tpu as pltpu
from jax.experimental.pallas import tpu_sc as plsc

N = 50000
E = 800000
KS = 5
K = 25
G = 64
F1 = 32
F2 = 64

NC = 2
NS = 16

EP = 802816
C2 = 64
CH2 = EP // NS // C2
CH1 = EP // (NC * NS) // C2
NP_ = 53248
NODE_PER_TILE = NP_ // (NC * NS)
ND = 50048

_mesh = plsc.VectorSubcoreMesh(core_axis_name="c", subcore_axis_name="s")


def _iota16():
    return lax.iota(jnp.int32, 16)


def _bcast_lane(v, lane):
    idx = jnp.full((16, 1), lane, dtype=jnp.int32)
    dn = lax.GatherDimensionNumbers(offset_dims=(), collapsed_slice_dims=(0,),
                                    start_index_map=(0,))
    return lax.gather(v, idx, dn, (1,), mode=lax.GatherScatterMode.PROMISE_IN_BOUNDS)


def _edge_weights(ea0v, ea1v):
    v0 = ea0v * (KS - 1.0)
    v1 = ea1v * (KS - 1.0)
    b0 = jnp.clip(v0.astype(jnp.int32), 0, KS - 2)
    b1 = jnp.clip(v1.astype(jnp.int32), 0, KS - 2)
    f0 = v0 - b0.astype(jnp.float32)
    f1 = v1 - b1.astype(jnp.float32)
    w00 = (1.0 - f0) * (1.0 - f1)
    w01 = (1.0 - f0) * f1
    w10 = f0 * (1.0 - f1)
    w11 = f0 * f1
    cell = b0 + 4 * b1
    return (w00, w10, w01, w11), cell


def _sc_chunk_prologue(src_h, dst_h, ea0_h, ea1_h, base, srcb, dstb, ea0b, ea1b,
                       gbuf, wbuf, rowoff):
    pltpu.sync_copy(src_h.at[pl.ds(base, C2)], srcb)
    pltpu.sync_copy(dst_h.at[pl.ds(base, C2)], dstb)
    pltpu.sync_copy(ea0_h.at[pl.ds(base, C2)], ea0b)
    pltpu.sync_copy(ea1_h.at[pl.ds(base, C2)], ea1b)
    for j in range(C2 // 16):
        ea0v = ea0b[pl.ds(j * 16, 16)]
        ea1v = ea1b[pl.ds(j * 16, 16)]
        srcv = srcb[pl.ds(j * 16, 16)]
        (wa, wb_, wc, wd), cell = _edge_weights(ea0v, ea1v)
        valid = (base + j * 16 + _iota16()) < E
        mv = jnp.where(valid, 1.0, 0.0)
        gbuf[0, pl.ds(j * 16, 16)] = rowoff + srcv * 16 + cell
        wbuf[pl.ds(j * 16, 16)] = wa * mv
        wbuf[pl.ds(C2 + j * 16, 16)] = wb_ * mv
        wbuf[pl.ds(2 * C2 + j * 16, 16)] = wc * mv
        wbuf[pl.ds(3 * C2 + j * 16, 16)] = wd * mv


def _sc_chunk_message(wbuf, dstb, grow, msgf, sidx):
    lanes = _iota16()

    def jloop(j, _):
        wv = [wbuf[pl.ds(r * C2 + j * 16, 16)] for r in range(4)]
        dv = dstb[pl.ds(j * 16, 16)]
        for e16 in range(16):
            eoff = j * 16 + e16
            m0 = jnp.zeros((16,), jnp.float32)
            m1 = jnp.zeros((16,), jnp.float32)
            for r in range(4):
                wb2 = _bcast_lane(wv[r], e16)
                m0 = m0 + wb2 * grow[eoff, pl.ds(r * 32, 16)]
                m1 = m1 + wb2 * grow[eoff, pl.ds(r * 32 + 16, 16)]
            msgf[pl.ds(eoff * F1, 16)] = m0
            msgf[pl.ds(eoff * F1 + 16, 16)] = m1
            db = _bcast_lane(dv, e16) * F1
            sidx[eoff // 4, pl.ds((eoff % 4) * 32, 16)] = db + lanes
            sidx[eoff // 4, pl.ds((eoff % 4) * 32 + 16, 16)] = db + 16 + lanes
        return 0
    lax.fori_loop(0, C2 // 16, jloop, 0)


def _sc_zero_zbuf(zbuf):
    def zb(j, _):
        zbuf[pl.ds(j * 16, 16)] = jnp.zeros((16,), jnp.float32)
        return 0
    lax.fori_loop(0, 391, zb, 0)


def _sc_zero_acc(zbuf, acc, s):
    W = ND * F1 // NS

    def zz(j, _):
        pltpu.sync_copy(zbuf, acc.at[pl.ds(s * W + j * 6256, 6256)])
        return 0
    lax.fori_loop(0, W // 6256, zz, 0)


def _sc_writeback_acc(zbuf, acc, out_h, c, s):
    W = ND * F1 // NS

    def wb(j, _):
        off = s * W + j * 6256
        pltpu.sync_copy(acc.at[pl.ds(off, 6256)], zbuf)
        pltpu.sync_copy(zbuf, out_h.at[pl.ds(c * ND * F1 + off, 6256)])
        return 0
    lax.fori_loop(0, W // 6256, wb, 0)


NB = C2 * F1 // 128


def _sc_scatter_msg(msgf, sidx, acc):
    for r in range(NB):
        pltpu.sync_copy(msgf.at[pl.ds(r * 128, 128)], acc.at[sidx.at[r]], add=True)


ND2 = 50176


@functools.partial(
    pl.kernel,
    out_type=[
        jax.ShapeDtypeStruct((NC * ND * F1,), jnp.float32),
        jax.ShapeDtypeStruct((NC * ND2,), jnp.float32),
        jax.ShapeDtypeStruct((NC * G,), jnp.float32),
    ],
    mesh=_mesh,
    scratch_types=[
        pltpu.VMEM((6256,), jnp.float32),
        pltpu.VMEM((C2,), jnp.int32),
        pltpu.VMEM((C2,), jnp.int32),
        pltpu.VMEM((C2,), jnp.float32),
        pltpu.VMEM((C2,), jnp.float32),
        pltpu.VMEM((4 * C2,), jnp.float32),
        pltpu.VMEM((1, C2), jnp.int32),
        pltpu.VMEM((1, C2), jnp.int32),
        pltpu.VMEM((C2,), jnp.float32),
        pltpu.VMEM((C2, 128), jnp.float32),
        pltpu.VMEM((C2 * F1,), jnp.float32),
        pltpu.VMEM((NB, 128), jnp.int32),
        pltpu.VMEM_SHARED((ND * F1,), jnp.float32),
        pltpu.VMEM_SHARED((ND2,), jnp.float32),
        pltpu.VMEM_SHARED((G,), jnp.float32),
        pltpu.SemaphoreType.DMA,
    ],
)
def _sc1(src_h, dst_h, ea0_h, ea1_h, t1_h, batch_h,
         o1_h, deg_h, cnt_h,
         zbuf, srcb, dstb, ea0b, ea1b, wbuf, gbuf, didx, sval, grow, msgf, sidx,
         acc, accD, accC, sem):
    c = lax.axis_index("c")
    s = lax.axis_index("s")
    t = c * NS + s

    _sc_zero_zbuf(zbuf)
    _sc_zero_acc(zbuf, acc, s)
    pltpu.sync_copy(zbuf.at[pl.ds(0, ND2 // NS)],
                    accD.at[pl.ds(s * (ND2 // NS), ND2 // NS)])

    @pl.when(s == 0)
    def _():
        pltpu.sync_copy(zbuf.at[pl.ds(0, G)], accC.at[pl.ds(0, G)])
    plsc.subcore_barrier()

    def chunk(i, _):
        base = t * (CH1 * C2) + i * C2
        _sc_chunk_prologue(src_h, dst_h, ea0_h, ea1_h, base,
                           srcb, dstb, ea0b, ea1b, gbuf, wbuf, 0)
        for j in range(C2 // 16):
            valid = (base + j * 16 + _iota16()) < E
            sval[pl.ds(j * 16, 16)] = jnp.where(valid, 1.0, 0.0)
            didx[0, pl.ds(j * 16, 16)] = dstb[pl.ds(j * 16, 16)]
        pltpu.async_copy(t1_h.at[gbuf.at[0]], grow, sem).wait()
        _sc_chunk_message(wbuf, dstb, grow, msgf, sidx)
        _sc_scatter_msg(msgf, sidx, acc)
        pltpu.sync_copy(sval, accD.at[didx.at[0]], add=True)
        return 0
    lax.fori_loop(0, CH1, chunk, 0)

    def nchunk(i, _):
        base = t * NODE_PER_TILE + i * C2
        pltpu.sync_copy(batch_h.at[pl.ds(base, C2)], srcb)
        for j in range(C2 // 16):
            valid = (base + j * 16 + _iota16()) < N
            sval[pl.ds(j * 16, 16)] = jnp.where(valid, 1.0, 0.0)
            didx[0, pl.ds(j * 16, 16)] = srcb[pl.ds(j * 16, 16)]
        pltpu.sync_copy(sval, accC.at[didx.at[0]], add=True)
        return 0
    lax.fori_loop(0, NODE_PER_TILE // C2, nchunk, 0)

    plsc.subcore_barrier()
    _sc_writeback_acc(zbuf, acc, o1_h, c, s)
    doff = s * (ND2 // NS)
    pltpu.sync_copy(accD.at[pl.ds(doff, ND2 // NS)], zbuf.at[pl.ds(0, ND2 // NS)])
    pltpu.sync_copy(zbuf.at[pl.ds(0, ND2 // NS)],
                    deg_h.at[pl.ds(c * ND2 + doff, ND2 // NS)])

    @pl.when(s == 0)
    def _():
        pltpu.sync_copy(accC.at[pl.ds(0, G)], zbuf.at[pl.ds(0, G)])
        pltpu.sync_copy(zbuf.at[pl.ds(0, G)], cnt_h.at[pl.ds(c * G, G)])


@functools.partial(
    pl.kernel,
    out_type=jax.ShapeDtypeStruct((NC * ND * F1,), jnp.float32),
    mesh=_mesh,
    scratch_types=[
        pltpu.VMEM((6256,), jnp.float32),
        pltpu.VMEM((C2,), jnp.int32),
        pltpu.VMEM((C2,), jnp.int32),
        pltpu.VMEM((C2,), jnp.float32),
        pltpu.VMEM((C2,), jnp.float32),
        pltpu.VMEM((4 * C2,), jnp.float32),
        pltpu.VMEM((1, C2), jnp.int32),
        pltpu.VMEM((C2, 128), jnp.float32),
        pltpu.VMEM((C2 * F1,), jnp.float32),
        pltpu.VMEM((NB, 128), jnp.int32),
        pltpu.VMEM_SHARED((ND * F1,), jnp.float32),
        pltpu.SemaphoreType.DMA,
    ],
)
def _sc2(src_h, dst_h, ea0_h, ea1_h, t2_h,
         o2_h,
         zbuf, srcb, dstb, ea0b, ea1b, wbuf, gbuf, grow, msgf, sidx,
         acc, sem):
    c = lax.axis_index("c")
    s = lax.axis_index("s")

    _sc_zero_zbuf(zbuf)
    _sc_zero_acc(zbuf, acc, s)
    plsc.subcore_barrier()

    coreoff = c * (N * 16)

    def chunk(i, _):
        base = s * (CH2 * C2) + i * C2
        _sc_chunk_prologue(src_h, dst_h, ea0_h, ea1_h, base,
                           srcb, dstb, ea0b, ea1b, gbuf, wbuf, coreoff)
        pltpu.async_copy(t2_h.at[gbuf.at[0]], grow, sem).wait()
        _sc_chunk_message(wbuf, dstb, grow, msgf, sidx)
        _sc_scatter_msg(msgf, sidx, acc)
        return 0
    lax.fori_loop(0, CH2, chunk, 0)

    plsc.subcore_barrier()
    _sc_writeback_acc(zbuf, acc, o2_h, c, s)


BA = 1000
GA = N // BA


def _tca0_body(x, w1q, t1_ref):
    xv = x[...]
    for cell in range(16):
        t1_ref[:, cell, :] = xv * w1q[:, cell * 128:(cell + 1) * 128]


_tca0 = pl.pallas_call(
    _tca0_body,
    grid=(GA,),
    in_specs=[
        pl.BlockSpec((BA, 1), lambda i: (i, 0)),
        pl.BlockSpec((1, 2048), lambda i: (0, 0)),
    ],
    out_specs=pl.BlockSpec((BA, 16, 128), lambda i: (i, 0, 0)),
    out_shape=jax.ShapeDtypeStruct((N, 16, 128), jnp.float32),
)


def _tca_body(o1a, o1b, d0, d1, x, r1, b1, w2a, w2b, h_ref, pab_ref):
    deg = jnp.maximum(d0[...] + d1[...], 1.0)
    o = (o1a[...] + o1b[...]) / deg
    o = o + x[...] * r1[...] + b1[...]
    h = jnp.where(o > 0, o, jnp.exp(o) - 1.0)
    h_ref[...] = h
    qa = jnp.dot(h, w2a[...], preferred_element_type=jnp.float32)
    qb = jnp.dot(h, w2b[...], preferred_element_type=jnp.float32)
    for cell in range(16):
        pab_ref[0, :, cell, :] = qa[:, cell * 128:(cell + 1) * 128]
        pab_ref[1, :, cell, :] = qb[:, cell * 128:(cell + 1) * 128]


_tca = pl.pallas_call(
    _tca_body,
    grid=(GA,),
    in_specs=[
        pl.BlockSpec((BA, F1), lambda i: (i, 0)),
        pl.BlockSpec((BA, F1), lambda i: (i, 0)),
        pl.BlockSpec((BA, 1), lambda i: (i, 0)),
        pl.BlockSpec((BA, 1), lambda i: (i, 0)),
        pl.BlockSpec((BA, 1), lambda i: (i, 0)),
        pl.BlockSpec((1, F1), lambda i: (0, 0)),
        pl.BlockSpec((1, F1), lambda i: (0, 0)),
        pl.BlockSpec((F1, 2048), lambda i: (0, 0)),
        pl.BlockSpec((F1, 2048), lambda i: (0, 0)),
    ],
    out_specs=[
        pl.BlockSpec((BA, F1), lambda i: (i, 0)),
        pl.BlockSpec((2, BA, 16, 128), lambda i: (0, i, 0, 0)),
    ],
    out_shape=[
        jax.ShapeDtypeStruct((N, F1), jnp.float32),
        jax.ShapeDtypeStruct((2, N, 16, 128), jnp.float32),
    ],
)


BB = 1000
GB = N // BB


def _tcb_body(oa, ob, h, d0, d1, bat, cnt2, r2, b2, wl1, bl1, wl2, bl2,
              out_ref, pooled):
    i = pl.program_id(0)

    @pl.when(i == 0)
    def _():
        pooled[...] = jnp.zeros_like(pooled)

    deg = jnp.maximum(d0[...] + d1[...], 1.0)
    o = jnp.concatenate([oa[...], ob[...]], axis=1) / deg
    o = o + jnp.dot(h[...], r2[...], preferred_element_type=jnp.float32) + b2[...]
    h2 = jnp.where(o > 0, o, jnp.exp(o) - 1.0)
    gids = lax.broadcasted_iota(jnp.int32, (BB, G), 1)
    oneh = (bat[...] == gids).astype(jnp.float32)
    pooled[...] += lax.dot_general(oneh, h2, (((0,), (0,)), ((), ())),
                                   preferred_element_type=jnp.float32)

    @pl.when(i == GB - 1)
    def _():
        cnt = jnp.maximum(cnt2[:, 0:1] + cnt2[:, 1:2], 1.0)
        p = pooled[...] / cnt
        hb = jnp.dot(p, wl1[...], preferred_element_type=jnp.float32) + bl1[...]
        hb = jnp.where(hb > 0, hb, jnp.exp(hb) - 1.0)
        lg = jnp.dot(hb, wl2[...], preferred_element_type=jnp.float32) + bl2[...]
        m = jnp.max(lg, axis=1, keepdims=True)
        sh = lg - m
        lse = jnp.log(jnp.sum(jnp.exp(sh), axis=1, keepdims=True))
        out_ref[...] = sh - lse


_tcb = pl.pallas_call(
    _tcb_body,
    grid=(GB,),
    in_specs=[
        pl.BlockSpec((BB, F1), lambda i: (i, 0)),
        pl.BlockSpec((BB, F1), lambda i: (i, 0)),
        pl.BlockSpec((BB, F1), lambda i: (i, 0)),
        pl.BlockSpec((BB, 1), lambda i: (i, 0)),
        pl.BlockSpec((BB, 1), lambda i: (i, 0)),
        pl.BlockSpec((BB, 1), lambda i: (i, 0)),
        pl.BlockSpec((G, 2), lambda i: (0, 0)),
        pl.BlockSpec((F1, F2), lambda i: (0, 0)),
        pl.BlockSpec((1, F2), lambda i: (0, 0)),
        pl.BlockSpec((F2, 128), lambda i: (0, 0)),
        pl.BlockSpec((1, 128), lambda i: (0, 0)),
        pl.BlockSpec((128, 10), lambda i: (0, 0)),
        pl.BlockSpec((1, 10), lambda i: (0, 0)),
    ],
    out_specs=pl.BlockSpec((G, 10), lambda i: (0, 0)),
    out_shape=jax.ShapeDtypeStruct((G, 10), jnp.float32),
    scratch_shapes=[pltpu.VMEM((G, F2), jnp.float32)],
)


@jax.jit
def kernel(x, edge_index, edge_attr, batch, W1, root1, b1, W2, root2, b2,
           Wl1, bl1, Wl2, bl2):
    src = edge_index[0].astype(jnp.int32)
    dst = edge_index[1].astype(jnp.int32)
    padE = EP - E
    srcp = jnp.concatenate([src, jnp.zeros((padE,), jnp.int32)])
    dstp = jnp.concatenate([dst, jnp.zeros((padE,), jnp.int32)])
    ea0 = jnp.concatenate([edge_attr[:, 0], jnp.zeros((padE,), jnp.float32)])
    ea1 = jnp.concatenate([edge_attr[:, 1], jnp.zeros((padE,), jnp.float32)])
    batchp = jnp.concatenate([batch.astype(jnp.int32),
                              jnp.zeros((NP_ - N,), jnp.int32)])

    kq = jnp.array([(cell % 4) + 5 * (cell // 4) + d
                    for cell in range(16) for d in (0, 1, 5, 6)], jnp.int32)
    w1q = W1[:, 0, :][kq].reshape(1, 2048)
    w2sel = W2[kq]
    w2a = w2sel[:, :, :F1].transpose(1, 0, 2).reshape(F1, 2048)
    w2b = w2sel[:, :, F1:].transpose(1, 0, 2).reshape(F1, 2048)

    t1 = _tca0(x, w1q)
    t1r = t1.reshape(N * 16, 128)

    o1p, degp, cntp = _sc1(srcp, dstp, ea0, ea1, t1r, batchp)
    o1a = o1p[:ND * F1].reshape(ND, F1)[:N]
    o1b = o1p[ND * F1:].reshape(ND, F1)[:N]
    d0 = degp[:N].reshape(N, 1)
    d1 = degp[ND2:ND2 + N].reshape(N, 1)
    cnt2 = cntp.reshape(NC, G).T

    h, pab = _tca(o1a, o1b, d0, d1, x, root1, b1.reshape(1, F1), w2a, w2b)

    t2r = pab.reshape(NC * N * 16, 128)
    o2p = _sc2(srcp, dstp, ea0, ea1, t2r)

    oa = o2p[:ND * F1].reshape(ND, F1)[:N]
    ob = o2p[ND * F1:].reshape(ND, F1)[:N]
    out = _tcb(oa, ob, h, d0, d1, batch.astype(jnp.int32).reshape(N, 1),
               cnt2, root2, b2.reshape(1, F2), Wl1, bl1.reshape(1, 128),
               Wl2, bl2.reshape(1, 10))
    return out


# speedup vs baseline: 4.3385x; 1.1882x over previous
import functools

import jax
import jax.numpy as jnp
from jax import lax
from jax.experimental import pallas as pl
from jax.experimental.pallas import tpu as pltpu
from jax.experimental.pallas import tpu_sc as plsc

N = 50000
E = 800000
KS = 5
K = 25
G = 64
F1 = 32
F2 = 64

NC = 2
NS = 16

EP = 802816
C2 = 64
CH2 = EP // NS // C2
CH1 = EP // (NC * NS) // C2
NP_ = 53248
NODE_PER_TILE = NP_ // (NC * NS)
ND = 50048

_mesh = plsc.VectorSubcoreMesh(core_axis_name="c", subcore_axis_name="s")


def _iota16():
    return lax.iota(jnp.int32, 16)


def _bcast_lane(v, lane):
    idx = jnp.full((16, 1), lane, dtype=jnp.int32)
    dn = lax.GatherDimensionNumbers(offset_dims=(), collapsed_slice_dims=(0,),
                                    start_index_map=(0,))
    return lax.gather(v, idx, dn, (1,), mode=lax.GatherScatterMode.PROMISE_IN_BOUNDS)


def _edge_weights(ea0v, ea1v):
    v0 = ea0v * (KS - 1.0)
    v1 = ea1v * (KS - 1.0)
    b0 = jnp.clip(v0.astype(jnp.int32), 0, KS - 2)
    b1 = jnp.clip(v1.astype(jnp.int32), 0, KS - 2)
    f0 = v0 - b0.astype(jnp.float32)
    f1 = v1 - b1.astype(jnp.float32)
    w00 = (1.0 - f0) * (1.0 - f1)
    w01 = (1.0 - f0) * f1
    w10 = f0 * (1.0 - f1)
    w11 = f0 * f1
    cell = b0 + 4 * b1
    return (w00, w10, w01, w11), cell


def _sc_chunk_prologue(src_h, dst_h, ea0_h, ea1_h, base, srcb, dstb, ea0b, ea1b,
                       gbuf, wbuf, rowoff):
    pltpu.sync_copy(src_h.at[pl.ds(base, C2)], srcb)
    pltpu.sync_copy(dst_h.at[pl.ds(base, C2)], dstb)
    pltpu.sync_copy(ea0_h.at[pl.ds(base, C2)], ea0b)
    pltpu.sync_copy(ea1_h.at[pl.ds(base, C2)], ea1b)
    for j in range(C2 // 16):
        ea0v = ea0b[pl.ds(j * 16, 16)]
        ea1v = ea1b[pl.ds(j * 16, 16)]
        srcv = srcb[pl.ds(j * 16, 16)]
        (wa, wb_, wc, wd), cell = _edge_weights(ea0v, ea1v)
        valid = (base + j * 16 + _iota16()) < E
        mv = jnp.where(valid, 1.0, 0.0)
        gbuf[0, pl.ds(j * 16, 16)] = rowoff + srcv * 16 + cell
        wbuf[pl.ds(j * 16, 16)] = wa * mv
        wbuf[pl.ds(C2 + j * 16, 16)] = wb_ * mv
        wbuf[pl.ds(2 * C2 + j * 16, 16)] = wc * mv
        wbuf[pl.ds(3 * C2 + j * 16, 16)] = wd * mv


def _sc_chunk_message(wbuf, dstb, grow, msgf, sidx):
    lanes = _iota16()

    def jloop(j, _):
        wv = [wbuf[pl.ds(r * C2 + j * 16, 16)] for r in range(4)]
        dv = dstb[pl.ds(j * 16, 16)]
        for e16 in range(16):
            eoff = j * 16 + e16
            m0 = jnp.zeros((16,), jnp.float32)
            m1 = jnp.zeros((16,), jnp.float32)
            for r in range(4):
                wb2 = _bcast_lane(wv[r], e16)
                m0 = m0 + wb2 * grow[eoff, pl.ds(r * 32, 16)]
                m1 = m1 + wb2 * grow[eoff, pl.ds(r * 32 + 16, 16)]
            msgf[pl.ds(eoff * F1, 16)] = m0
            msgf[pl.ds(eoff * F1 + 16, 16)] = m1
            db = _bcast_lane(dv, e16) * F1
            sidx[pl.ds(eoff * F1, 16)] = db + lanes
            sidx[pl.ds(eoff * F1 + 16, 16)] = db + 16 + lanes
        return 0
    lax.fori_loop(0, C2 // 16, jloop, 0)


def _sc_zero_zbuf(zbuf):
    def zb(j, _):
        zbuf[pl.ds(j * 16, 16)] = jnp.zeros((16,), jnp.float32)
        return 0
    lax.fori_loop(0, 391, zb, 0)


def _sc_zero_acc(zbuf, acc, s):
    W = ND * F1 // NS

    def zz(j, _):
        pltpu.sync_copy(zbuf, acc.at[pl.ds(s * W + j * 6256, 6256)])
        return 0
    lax.fori_loop(0, W // 6256, zz, 0)


def _sc_writeback_acc(zbuf, acc, out_h, c, s):
    W = ND * F1 // NS

    def wb(j, _):
        off = s * W + j * 6256
        pltpu.sync_copy(acc.at[pl.ds(off, 6256)], zbuf)
        pltpu.sync_copy(zbuf, out_h.at[pl.ds(c * ND * F1 + off, 6256)])
        return 0
    lax.fori_loop(0, W // 6256, wb, 0)


NB = C2 * F1 // 128


def _sc_scatter_msg(msgf, sidx, acc):
    pltpu.sync_copy(msgf, acc.at[sidx], add=True)


ND2 = 50176


@functools.partial(
    pl.kernel,
    out_type=[
        jax.ShapeDtypeStruct((NC * ND * F1,), jnp.float32),
        jax.ShapeDtypeStruct((NC * ND2,), jnp.float32),
        jax.ShapeDtypeStruct((NC * G,), jnp.float32),
    ],
    mesh=_mesh,
    scratch_types=[
        pltpu.VMEM((6256,), jnp.float32),
        pltpu.VMEM((C2,), jnp.int32),
        pltpu.VMEM((C2,), jnp.int32),
        pltpu.VMEM((C2,), jnp.float32),
        pltpu.VMEM((C2,), jnp.float32),
        pltpu.VMEM((4 * C2,), jnp.float32),
        pltpu.VMEM((1, C2), jnp.int32),
        pltpu.VMEM((1, C2), jnp.int32),
        pltpu.VMEM((C2,), jnp.float32),
        pltpu.VMEM((C2, 128), jnp.float32),
        pltpu.VMEM((C2 * F1,), jnp.float32),
        pltpu.VMEM((C2 * F1,), jnp.int32),
        pltpu.VMEM_SHARED((ND * F1,), jnp.float32),
        pltpu.VMEM_SHARED((ND2,), jnp.float32),
        pltpu.VMEM_SHARED((G,), jnp.float32),
        pltpu.SemaphoreType.DMA,
    ],
)
def _sc1(src_h, dst_h, ea0_h, ea1_h, t1_h, batch_h,
         o1_h, deg_h, cnt_h,
         zbuf, srcb, dstb, ea0b, ea1b, wbuf, gbuf, didx, sval, grow, msgf, sidx,
         acc, accD, accC, sem):
    c = lax.axis_index("c")
    s = lax.axis_index("s")
    t = c * NS + s

    _sc_zero_zbuf(zbuf)
    _sc_zero_acc(zbuf, acc, s)
    pltpu.sync_copy(zbuf.at[pl.ds(0, ND2 // NS)],
                    accD.at[pl.ds(s * (ND2 // NS), ND2 // NS)])

    @pl.when(s == 0)
    def _():
        pltpu.sync_copy(zbuf.at[pl.ds(0, G)], accC.at[pl.ds(0, G)])
    plsc.subcore_barrier()

    def chunk(i, _):
        base = t * (CH1 * C2) + i * C2
        _sc_chunk_prologue(src_h, dst_h, ea0_h, ea1_h, base,
                           srcb, dstb, ea0b, ea1b, gbuf, wbuf, 0)
        for j in range(C2 // 16):
            valid = (base + j * 16 + _iota16()) < E
            sval[pl.ds(j * 16, 16)] = jnp.where(valid, 1.0, 0.0)
            didx[0, pl.ds(j * 16, 16)] = dstb[pl.ds(j * 16, 16)]
        pltpu.async_copy(t1_h.at[gbuf.at[0]], grow, sem).wait()
        _sc_chunk_message(wbuf, dstb, grow, msgf, sidx)
        _sc_scatter_msg(msgf, sidx, acc)
        pltpu.sync_copy(sval, accD.at[didx.at[0]], add=True)
        return 0
    lax.fori_loop(0, CH1, chunk, 0)

    def nchunk(i, _):
        base = t * NODE_PER_TILE + i * C2
        pltpu.sync_copy(batch_h.at[pl.ds(base, C2)], srcb)
        for j in range(C2 // 16):
            valid = (base + j * 16 + _iota16()) < N
            sval[pl.ds(j * 16, 16)] = jnp.where(valid, 1.0, 0.0)
            didx[0, pl.ds(j * 16, 16)] = srcb[pl.ds(j * 16, 16)]
        pltpu.sync_copy(sval, accC.at[didx.at[0]], add=True)
        return 0
    lax.fori_loop(0, NODE_PER_TILE // C2, nchunk, 0)

    plsc.subcore_barrier()
    _sc_writeback_acc(zbuf, acc, o1_h, c, s)
    doff = s * (ND2 // NS)
    pltpu.sync_copy(accD.at[pl.ds(doff, ND2 // NS)], zbuf.at[pl.ds(0, ND2 // NS)])
    pltpu.sync_copy(zbuf.at[pl.ds(0, ND2 // NS)],
                    deg_h.at[pl.ds(c * ND2 + doff, ND2 // NS)])

    @pl.when(s == 0)
    def _():
        pltpu.sync_copy(accC.at[pl.ds(0, G)], zbuf.at[pl.ds(0, G)])
        pltpu.sync_copy(zbuf.at[pl.ds(0, G)], cnt_h.at[pl.ds(c * G, G)])


@functools.partial(
    pl.kernel,
    out_type=jax.ShapeDtypeStruct((NC * ND * F1,), jnp.float32),
    mesh=_mesh,
    scratch_types=[
        pltpu.VMEM((6256,), jnp.float32),
        pltpu.VMEM((C2,), jnp.int32),
        pltpu.VMEM((C2,), jnp.int32),
        pltpu.VMEM((C2,), jnp.float32),
        pltpu.VMEM((C2,), jnp.float32),
        pltpu.VMEM((4 * C2,), jnp.float32),
        pltpu.VMEM((1, C2), jnp.int32),
        pltpu.VMEM((C2, 128), jnp.float32),
        pltpu.VMEM((C2 * F1,), jnp.float32),
        pltpu.VMEM((C2 * F1,), jnp.int32),
        pltpu.VMEM_SHARED((ND * F1,), jnp.float32),
        pltpu.SemaphoreType.DMA,
    ],
)
def _sc2(src_h, dst_h, ea0_h, ea1_h, t2_h,
         o2_h,
         zbuf, srcb, dstb, ea0b, ea1b, wbuf, gbuf, grow, msgf, sidx,
         acc, sem):
    c = lax.axis_index("c")
    s = lax.axis_index("s")

    _sc_zero_zbuf(zbuf)
    _sc_zero_acc(zbuf, acc, s)
    plsc.subcore_barrier()

    coreoff = c * (N * 16)

    def chunk(i, _):
        base = s * (CH2 * C2) + i * C2
        _sc_chunk_prologue(src_h, dst_h, ea0_h, ea1_h, base,
                           srcb, dstb, ea0b, ea1b, gbuf, wbuf, coreoff)
        pltpu.async_copy(t2_h.at[gbuf.at[0]], grow, sem).wait()
        _sc_chunk_message(wbuf, dstb, grow, msgf, sidx)
        _sc_scatter_msg(msgf, sidx, acc)
        return 0
    lax.fori_loop(0, CH2, chunk, 0)

    plsc.subcore_barrier()
    _sc_writeback_acc(zbuf, acc, o2_h, c, s)


BA = 1000
GA = N // BA


def _tca0_body(x, w1q, t1_ref):
    xv = x[...]
    for cell in range(16):
        t1_ref[:, cell, :] = xv * w1q[:, cell * 128:(cell + 1) * 128]


_tca0 = pl.pallas_call(
    _tca0_body,
    grid=(GA,),
    in_specs=[
        pl.BlockSpec((BA, 1), lambda i: (i, 0)),
        pl.BlockSpec((1, 2048), lambda i: (0, 0)),
    ],
    out_specs=pl.BlockSpec((BA, 16, 128), lambda i: (i, 0, 0)),
    out_shape=jax.ShapeDtypeStruct((N, 16, 128), jnp.float32),
)


def _tca_body(o1a, o1b, d0, d1, x, r1, b1, w2a, w2b, h_ref, pab_ref):
    deg = jnp.maximum(d0[...] + d1[...], 1.0)
    o = (o1a[...] + o1b[...]) / deg
    o = o + x[...] * r1[...] + b1[...]
    h = jnp.where(o > 0, o, jnp.exp(o) - 1.0)
    h_ref[...] = h
    qa = jnp.dot(h, w2a[...], preferred_element_type=jnp.float32)
    qb = jnp.dot(h, w2b[...], preferred_element_type=jnp.float32)
    for cell in range(16):
        pab_ref[0, :, cell, :] = qa[:, cell * 128:(cell + 1) * 128]
        pab_ref[1, :, cell, :] = qb[:, cell * 128:(cell + 1) * 128]


_tca = pl.pallas_call(
    _tca_body,
    grid=(GA,),
    in_specs=[
        pl.BlockSpec((BA, F1), lambda i: (i, 0)),
        pl.BlockSpec((BA, F1), lambda i: (i, 0)),
        pl.BlockSpec((BA, 1), lambda i: (i, 0)),
        pl.BlockSpec((BA, 1), lambda i: (i, 0)),
        pl.BlockSpec((BA, 1), lambda i: (i, 0)),
        pl.BlockSpec((1, F1), lambda i: (0, 0)),
        pl.BlockSpec((1, F1), lambda i: (0, 0)),
        pl.BlockSpec((F1, 2048), lambda i: (0, 0)),
        pl.BlockSpec((F1, 2048), lambda i: (0, 0)),
    ],
    out_specs=[
        pl.BlockSpec((BA, F1), lambda i: (i, 0)),
        pl.BlockSpec((2, BA, 16, 128), lambda i: (0, i, 0, 0)),
    ],
    out_shape=[
        jax.ShapeDtypeStruct((N, F1), jnp.float32),
        jax.ShapeDtypeStruct((2, N, 16, 128), jnp.float32),
    ],
)


BB = 1000
GB = N // BB


def _tcb_body(oa, ob, h, d0, d1, bat, cnt2, r2, b2, wl1, bl1, wl2, bl2,
              out_ref, pooled):
    i = pl.program_id(0)

    @pl.when(i == 0)
    def _():
        pooled[...] = jnp.zeros_like(pooled)

    deg = jnp.maximum(d0[...] + d1[...], 1.0)
    o = jnp.concatenate([oa[...], ob[...]], axis=1) / deg
    o = o + jnp.dot(h[...], r2[...], preferred_element_type=jnp.float32) + b2[...]
    h2 = jnp.where(o > 0, o, jnp.exp(o) - 1.0)
    gids = lax.broadcasted_iota(jnp.int32, (BB, G), 1)
    oneh = (bat[...] == gids).astype(jnp.float32)
    pooled[...] += lax.dot_general(oneh, h2, (((0,), (0,)), ((), ())),
                                   preferred_element_type=jnp.float32)

    @pl.when(i == GB - 1)
    def _():
        cnt = jnp.maximum(cnt2[:, 0:1] + cnt2[:, 1:2], 1.0)
        p = pooled[...] / cnt
        hb = jnp.dot(p, wl1[...], preferred_element_type=jnp.float32) + bl1[...]
        hb = jnp.where(hb > 0, hb, jnp.exp(hb) - 1.0)
        lg = jnp.dot(hb, wl2[...], preferred_element_type=jnp.float32) + bl2[...]
        m = jnp.max(lg, axis=1, keepdims=True)
        sh = lg - m
        lse = jnp.log(jnp.sum(jnp.exp(sh), axis=1, keepdims=True))
        out_ref[...] = sh - lse


_tcb = pl.pallas_call(
    _tcb_body,
    grid=(GB,),
    in_specs=[
        pl.BlockSpec((BB, F1), lambda i: (i, 0)),
        pl.BlockSpec((BB, F1), lambda i: (i, 0)),
        pl.BlockSpec((BB, F1), lambda i: (i, 0)),
        pl.BlockSpec((BB, 1), lambda i: (i, 0)),
        pl.BlockSpec((BB, 1), lambda i: (i, 0)),
        pl.BlockSpec((BB, 1), lambda i: (i, 0)),
        pl.BlockSpec((G, 2), lambda i: (0, 0)),
        pl.BlockSpec((F1, F2), lambda i: (0, 0)),
        pl.BlockSpec((1, F2), lambda i: (0, 0)),
        pl.BlockSpec((F2, 128), lambda i: (0, 0)),
        pl.BlockSpec((1, 128), lambda i: (0, 0)),
        pl.BlockSpec((128, 10), lambda i: (0, 0)),
        pl.BlockSpec((1, 10), lambda i: (0, 0)),
    ],
    out_specs=pl.BlockSpec((G, 10), lambda i: (0, 0)),
    out_shape=jax.ShapeDtypeStruct((G, 10), jnp.float32),
    scratch_shapes=[pltpu.VMEM((G, F2), jnp.float32)],
)


@jax.jit
def kernel(x, edge_index, edge_attr, batch, W1, root1, b1, W2, root2, b2,
           Wl1, bl1, Wl2, bl2):
    src = edge_index[0].astype(jnp.int32)
    dst = edge_index[1].astype(jnp.int32)
    padE = EP - E
    srcp = jnp.concatenate([src, jnp.zeros((padE,), jnp.int32)])
    dstp = jnp.concatenate([dst, jnp.zeros((padE,), jnp.int32)])
    ea0 = jnp.concatenate([edge_attr[:, 0], jnp.zeros((padE,), jnp.float32)])
    ea1 = jnp.concatenate([edge_attr[:, 1], jnp.zeros((padE,), jnp.float32)])
    batchp = jnp.concatenate([batch.astype(jnp.int32),
                              jnp.zeros((NP_ - N,), jnp.int32)])

    kq = jnp.array([(cell % 4) + 5 * (cell // 4) + d
                    for cell in range(16) for d in (0, 1, 5, 6)], jnp.int32)
    w1q = W1[:, 0, :][kq].reshape(1, 2048)
    w2sel = W2[kq]
    w2a = w2sel[:, :, :F1].transpose(1, 0, 2).reshape(F1, 2048)
    w2b = w2sel[:, :, F1:].transpose(1, 0, 2).reshape(F1, 2048)

    t1 = _tca0(x, w1q)
    t1r = t1.reshape(N * 16, 128)

    o1p, degp, cntp = _sc1(srcp, dstp, ea0, ea1, t1r, batchp)
    o1a = o1p[:ND * F1].reshape(ND, F1)[:N]
    o1b = o1p[ND * F1:].reshape(ND, F1)[:N]
    d0 = degp[:N].reshape(N, 1)
    d1 = degp[ND2:ND2 + N].reshape(N, 1)
    cnt2 = cntp.reshape(NC, G).T

    h, pab = _tca(o1a, o1b, d0, d1, x, root1, b1.reshape(1, F1), w2a, w2b)

    t2r = pab.reshape(NC * N * 16, 128)
    o2p = _sc2(srcp, dstp, ea0, ea1, t2r)

    oa = o2p[:ND * F1].reshape(ND, F1)[:N]
    ob = o2p[ND * F1:].reshape(ND, F1)[:N]
    out = _tcb(oa, ob, h, d0, d1, batch.astype(jnp.int32).reshape(N, 1),
               cnt2, root2, b2.reshape(1, F2), Wl1, bl1.reshape(1, 128),
               Wl2, bl2.reshape(1, 10))
    return out
